# Initial kernel scaffold; baseline (speedup 1.0000x reference)
import jax
import jax.numpy as jnp
from jax.experimental import pallas as pl


def kernel(idx, wte, w_unembed):
    raise NotImplementedError("write your pallas kernel here")


# speedup vs baseline: 5.0746x; 5.0746x over previous
import functools

import jax
import jax.numpy as jnp
from jax import lax
from jax.experimental import pallas as pl
from jax.experimental.pallas import tpu as pltpu
from jax.experimental.pallas import tpu_sc as plsc

_VOCAB = 50257
_D = 768
_K = 50
_TV = 2048
_NV = (_VOCAB + _TV - 1) // _TV
_NEG = -1e30


def _sc_gather(last_idx, wte):
    B = last_idx.shape[0]
    D = wte.shape[1]
    nwork = 8
    bpw = B // nwork
    mesh = plsc.VectorSubcoreMesh(core_axis_name="c", subcore_axis_name="s")

    @functools.partial(
        pl.kernel,
        mesh=mesh,
        out_type=jax.ShapeDtypeStruct((B, D), jnp.float32),
        scratch_types=[
            pltpu.VMEM((bpw,), jnp.int32),
            pltpu.VMEM((bpw, D), jnp.float32),
            pltpu.SemaphoreType.DMA,
        ],
    )
    def gather_kernel(idx_hbm, table_hbm, out_hbm, idx_v, rows_v, sem):
        nc = 2
        wid = lax.axis_index("s") * nc + lax.axis_index("c")

        @pl.when(wid < nwork)
        def _():
            base = wid * bpw
            pltpu.sync_copy(idx_hbm.at[pl.ds(base, bpw)], idx_v)
            pltpu.async_copy(table_hbm.at[idx_v], rows_v, sem).wait()
            pltpu.sync_copy(rows_v, out_hbm.at[pl.ds(base, bpw)])

    return gather_kernel(last_idx, wte)


def _tc_body(h_ref, w_ref, g_ref, tok_ref, probs_ref, idx_ref, logits_ref):
    v = pl.program_id(0)
    B = h_ref.shape[0]

    lane = lax.broadcasted_iota(jnp.int32, (B, _TV), 1)
    cols = lane + v * _TV
    lt = jnp.dot(h_ref[...], w_ref[...], preferred_element_type=jnp.float32)
    lt = jnp.where(cols < _VOCAB, lt, _NEG)
    logits_ref[v] = lt

    @pl.when(v == _NV - 1)
    def _finalize():
        neg_inf = jnp.float32(-jnp.inf)
        log2e = jnp.float32(1.4426950408889634)

        def mx(j, m):
            return jnp.maximum(m, jnp.max(logits_ref[j], axis=1, keepdims=True))

        m = lax.fori_loop(0, _NV, mx, jnp.full((B, 1), neg_inf, jnp.float32))

        def se(j, acc):
            ej = jnp.exp2(log2e * (logits_ref[j] - m))
            logits_ref[j] = ej
            for t in range(_TV // 128):
                acc = acc + ej[:, t * 128:(t + 1) * 128]
            return acc

        acc = lax.fori_loop(0, _NV, se, jnp.zeros((B, 128), jnp.float32))
        s = jnp.sum(acc, axis=-1, keepdims=True)
        r = pl.reciprocal(s, approx=True)

        def pdiv(j, _):
            cj = lane + j * _TV
            pj = logits_ref[j] * r
            logits_ref[j] = jnp.where(cj < _VOCAB, pj, -1.0)
            return 0

        lax.fori_loop(0, _NV, pdiv, 0)

        kiota = lax.broadcasted_iota(jnp.int32, (B, _K), 1)

        def pick(k, carry):
            vals, colsb, prev = carry

            def scan(j, c):
                cm, ca = c
                xj = logits_ref[j]
                xj = jnp.where(lane == prev - j * _TV, neg_inf, xj)
                logits_ref[j] = xj
                mj = jnp.max(xj, axis=1, keepdims=True)
                aj = jnp.min(jnp.where(xj == mj, lane, _TV), axis=1,
                             keepdims=True) + j * _TV
                better = mj > cm
                return jnp.maximum(cm, mj), jnp.where(better, aj, ca)

            cm, ca = lax.fori_loop(
                0, _NV, scan,
                (jnp.full((B, 1), neg_inf, jnp.float32),
                 jnp.zeros((B, 1), jnp.int32)))
            sel = kiota == k
            vals = jnp.where(sel, cm, vals)
            colsb = jnp.where(sel, ca, colsb)
            return vals, colsb, ca

        vals, colsb, _ = lax.fori_loop(
            0, _K, pick,
            (jnp.zeros((B, _K), jnp.float32),
             jnp.zeros((B, _K), jnp.int32),
             jnp.full((B, 1), -1, jnp.int32)))

        probs = vals
        y = jnp.log(probs + 1e-20) + g_ref[...]
        ym = jnp.max(y, axis=1, keepdims=True)
        ix = jnp.min(jnp.where(y == ym, kiota, _K), axis=1, keepdims=True)
        tok = jnp.sum(jnp.where(kiota == ix, colsb, 0), axis=1, keepdims=True)

        tok_ref[...] = tok
        probs_ref[...] = probs
        idx_ref[...] = colsb


def _tc_call(h, w_unembed, g):
    B = h.shape[0]
    return pl.pallas_call(
        _tc_body,
        grid=(_NV,),
        in_specs=[
            pl.BlockSpec((B, _D), lambda v: (0, 0)),
            pl.BlockSpec((_D, _TV), lambda v: (0, v)),
            pl.BlockSpec((B, _K), lambda v: (0, 0)),
        ],
        out_specs=[
            pl.BlockSpec((B, 1), lambda v: (0, 0)),
            pl.BlockSpec((B, _K), lambda v: (0, 0)),
            pl.BlockSpec((B, _K), lambda v: (0, 0)),
        ],
        out_shape=[
            jax.ShapeDtypeStruct((B, 1), jnp.int32),
            jax.ShapeDtypeStruct((B, _K), jnp.float32),
            jax.ShapeDtypeStruct((B, _K), jnp.int32),
        ],
        scratch_shapes=[pltpu.VMEM((_NV, B, _TV), jnp.float32)],
        compiler_params=pltpu.CompilerParams(
            dimension_semantics=("arbitrary",)),
    )(h, w_unembed, g)


def kernel(idx, wte, w_unembed):
    last_idx = idx[:, -1]
    h = _sc_gather(last_idx, wte)
    g = jax.random.gumbel(jax.random.key(42), (idx.shape[0], _K), jnp.float32)
    next_token, topk_probs, topk_indices = _tc_call(h, w_unembed, g)
    return next_token, topk_probs, topk_indices


# speedup vs baseline: 9.3162x; 1.8358x over previous
import functools

import jax
import jax.numpy as jnp
from jax import lax
from jax.experimental import pallas as pl
from jax.experimental.pallas import tpu as pltpu
from jax.experimental.pallas import tpu_sc as plsc

_VOCAB = 50257
_D = 768
_K = 50
_TV = 2048
_NV = (_VOCAB + _TV - 1) // _TV
_NEG = -1e30


def _sc_gather(last_idx, wte):
    B = last_idx.shape[0]
    D = wte.shape[1]
    nwork = 8
    bpw = B // nwork
    mesh = plsc.VectorSubcoreMesh(core_axis_name="c", subcore_axis_name="s")

    @functools.partial(
        pl.kernel,
        mesh=mesh,
        out_type=jax.ShapeDtypeStruct((B, D), jnp.float32),
        scratch_types=[
            pltpu.VMEM((bpw,), jnp.int32),
            pltpu.VMEM((bpw, D), jnp.float32),
            pltpu.SemaphoreType.DMA,
        ],
    )
    def gather_kernel(idx_hbm, table_hbm, out_hbm, idx_v, rows_v, sem):
        nc = 2
        wid = lax.axis_index("s") * nc + lax.axis_index("c")

        @pl.when(wid < nwork)
        def _():
            base = wid * bpw
            pltpu.sync_copy(idx_hbm.at[pl.ds(base, bpw)], idx_v)
            pltpu.async_copy(table_hbm.at[idx_v], rows_v, sem).wait()
            pltpu.sync_copy(rows_v, out_hbm.at[pl.ds(base, bpw)])

    return gather_kernel(last_idx, wte)


def _tc_body(h_ref, w_ref, g_ref, tok_ref, probs_ref, idx_ref, logits_ref,
             m_ref):
    v = pl.program_id(0)
    B = h_ref.shape[0]

    lane = lax.broadcasted_iota(jnp.int32, (B, _TV), 1)
    cols = lane + v * _TV
    lt = jnp.dot(h_ref[...], w_ref[...], preferred_element_type=jnp.float32)
    lt = jnp.where(cols < _VOCAB, lt, _NEG)
    logits_ref[v] = lt
    ltmax = jnp.max(lt, axis=1, keepdims=True)

    @pl.when(v == 0)
    def _():
        m_ref[...] = ltmax

    @pl.when(v > 0)
    def _():
        m_ref[...] = jnp.maximum(m_ref[...], ltmax)

    @pl.when(v == _NV - 1)
    def _finalize():
        neg_inf = jnp.float32(-jnp.inf)
        log2e = jnp.float32(1.4426950408889634)
        m = m_ref[...]

        def se(j, acc):
            ej = jnp.exp2(log2e * (logits_ref[j] - m))
            logits_ref[j] = ej
            for t in range(_TV // 128):
                acc = acc + ej[:, t * 128:(t + 1) * 128]
            return acc

        acc = lax.fori_loop(0, _NV, se, jnp.zeros((B, 128), jnp.float32))
        c = acc[:, 0:8]
        for t in range(1, 16):
            c = c + acc[:, 8 * t:8 * (t + 1)]
        u = c[:, 0:4] + c[:, 4:8]
        w = u[:, 0:2] + u[:, 2:4]
        s = w[:, 0:1] + w[:, 1:2]
        r = pl.reciprocal(s, approx=True)

        kiota = lax.broadcasted_iota(jnp.int32, (B, _K), 1)

        _DEP = 16
        lane32 = lax.broadcasted_iota(jnp.int32, (B, 32), 1)

        def build(j, carry):
            rs, cs = carry
            cj = lane + j * _TV
            cur = jnp.where(cj < _VOCAB, logits_ref[j] * r, -1.0)
            logits_ref[j] = cur
            rs2, cs2 = [], []
            for d in range(_DEP):
                mj = jnp.max(cur, axis=1, keepdims=True)
                aj = jnp.min(jnp.where(cur == mj, lane, _TV), axis=1,
                             keepdims=True)
                gcol = aj + j * _TV
                rs2.append(jnp.where(lane32 == j, mj, rs[d]))
                cs2.append(jnp.where(lane32 == j, gcol, cs[d]))
                if d + 1 < _DEP:
                    cur = jnp.where(lane == aj, neg_inf, cur)
            return tuple(rs2), tuple(cs2)

        rs0 = tuple(jnp.full((B, 32), neg_inf, jnp.float32)
                    for _ in range(_DEP))
        cs0 = tuple(jnp.zeros((B, 32), jnp.int32) for _ in range(_DEP))
        rs, cs = lax.fori_loop(0, _NV, build, (rs0, cs0))

        def pick_fast(k, carry):
            vals, colsb, cnt = carry
            cur = jnp.full((B, 32), neg_inf, jnp.float32)
            acol = jnp.zeros((B, 32), jnp.int32)
            for d in range(_DEP):
                hit = cnt == d
                cur = jnp.where(hit, rs[d], cur)
                acol = jnp.where(hit, cs[d], acol)
            cur = jnp.where(lane32 < _NV, cur, neg_inf)
            cm = jnp.max(cur, axis=1, keepdims=True)
            cidx = jnp.min(jnp.where(cur == cm, lane32, 32), axis=1,
                           keepdims=True)
            hitl = lane32 == cidx
            ca = jnp.sum(jnp.where(hitl, acol, 0), axis=1, keepdims=True)
            cnt = jnp.where(hitl, cnt + 1, cnt)
            sel = kiota == k
            vals = jnp.where(sel, cm, vals)
            colsb = jnp.where(sel, ca, colsb)
            return vals, colsb, cnt

        vals_f, colsb_f, cnt = lax.fori_loop(
            0, _K, pick_fast,
            (jnp.zeros((B, _K), jnp.float32),
             jnp.zeros((B, _K), jnp.int32),
             jnp.zeros((B, 32), jnp.int32)))
        overflow = jnp.any(cnt >= _DEP)

        def pick_slow(_):
            def pick(k, carry):
                vals, colsb, prev = carry

                def scan(j, c):
                    cm, ca = c
                    xj = logits_ref[j]
                    xj = jnp.where(lane == prev - j * _TV, neg_inf, xj)
                    logits_ref[j] = xj
                    mj = jnp.max(xj, axis=1, keepdims=True)
                    aj = jnp.min(jnp.where(xj == mj, lane, _TV), axis=1,
                                 keepdims=True) + j * _TV
                    better = mj > cm
                    return jnp.maximum(cm, mj), jnp.where(better, aj, ca)

                cm, ca = lax.fori_loop(
                    0, _NV, scan,
                    (jnp.full((B, 1), neg_inf, jnp.float32),
                     jnp.zeros((B, 1), jnp.int32)))
                sel = kiota == k
                vals = jnp.where(sel, cm, vals)
                colsb = jnp.where(sel, ca, colsb)
                return vals, colsb, ca

            vals, colsb, _ = lax.fori_loop(
                0, _K, pick,
                (jnp.zeros((B, _K), jnp.float32),
                 jnp.zeros((B, _K), jnp.int32),
                 jnp.full((B, 1), -1, jnp.int32)))
            return vals, colsb

        vals, colsb = lax.cond(
            overflow, pick_slow, lambda _: (vals_f, colsb_f), 0)

        probs = vals
        y = jnp.log(probs + 1e-20) + g_ref[...]
        ym = jnp.max(y, axis=1, keepdims=True)
        ix = jnp.min(jnp.where(y == ym, kiota, _K), axis=1, keepdims=True)
        tok = jnp.sum(jnp.where(kiota == ix, colsb, 0), axis=1, keepdims=True)

        tok_ref[...] = tok
        probs_ref[...] = probs
        idx_ref[...] = colsb


def _tc_call(h, w_unembed, g):
    B = h.shape[0]
    return pl.pallas_call(
        _tc_body,
        grid=(_NV,),
        in_specs=[
            pl.BlockSpec((B, _D), lambda v: (0, 0)),
            pl.BlockSpec((_D, _TV), lambda v: (0, v)),
            pl.BlockSpec((B, _K), lambda v: (0, 0)),
        ],
        out_specs=[
            pl.BlockSpec((B, 1), lambda v: (0, 0)),
            pl.BlockSpec((B, _K), lambda v: (0, 0)),
            pl.BlockSpec((B, _K), lambda v: (0, 0)),
        ],
        out_shape=[
            jax.ShapeDtypeStruct((B, 1), jnp.int32),
            jax.ShapeDtypeStruct((B, _K), jnp.float32),
            jax.ShapeDtypeStruct((B, _K), jnp.int32),
        ],
        scratch_shapes=[pltpu.VMEM((_NV, B, _TV), jnp.float32),
                        pltpu.VMEM((B, 1), jnp.float32)],
        compiler_params=pltpu.CompilerParams(
            dimension_semantics=("arbitrary",)),
    )(h, w_unembed, g)


def kernel(idx, wte, w_unembed):
    last_idx = idx[:, -1]
    h = _sc_gather(last_idx, wte)
    g = jax.random.gumbel(jax.random.key(42), (idx.shape[0], _K), jnp.float32)
    next_token, topk_probs, topk_indices = _tc_call(h, w_unembed, g)
    return next_token, topk_probs, topk_indices


# speedup vs baseline: 10.1678x; 1.0914x over previous
import functools

import jax
import jax.numpy as jnp
from jax import lax
from jax.experimental import pallas as pl
from jax.experimental.pallas import tpu as pltpu
from jax.experimental.pallas import tpu_sc as plsc

_VOCAB = 50257
_D = 768
_K = 50
_TV = 2048
_NV = (_VOCAB + _TV - 1) // _TV
_NEG = -1e30


def _sc_gather(last_idx, wte):
    B = last_idx.shape[0]
    D = wte.shape[1]
    nwork = 8
    bpw = B // nwork
    mesh = plsc.VectorSubcoreMesh(core_axis_name="c", subcore_axis_name="s")

    @functools.partial(
        pl.kernel,
        mesh=mesh,
        out_type=jax.ShapeDtypeStruct((B, D), jnp.float32),
        scratch_types=[
            pltpu.VMEM((bpw,), jnp.int32),
            pltpu.VMEM((bpw, D), jnp.float32),
            pltpu.SemaphoreType.DMA,
        ],
    )
    def gather_kernel(idx_hbm, table_hbm, out_hbm, idx_v, rows_v, sem):
        nc = 2
        wid = lax.axis_index("s") * nc + lax.axis_index("c")

        @pl.when(wid < nwork)
        def _():
            base = wid * bpw
            pltpu.sync_copy(idx_hbm.at[pl.ds(base, bpw)], idx_v)
            pltpu.async_copy(table_hbm.at[idx_v], rows_v, sem).wait()
            pltpu.sync_copy(rows_v, out_hbm.at[pl.ds(base, bpw)])

    return gather_kernel(last_idx, wte)


def _tc_body(h_ref, w_ref, g_ref, tok_ref, probs_ref, idx_ref, logits_ref,
             m_ref):
    v = pl.program_id(0)
    B = h_ref.shape[0]

    lane = lax.broadcasted_iota(jnp.int32, (B, _TV), 1)
    cols = lane + v * _TV
    lt = jnp.dot(h_ref[...], w_ref[...], preferred_element_type=jnp.float32)
    lt = jnp.where(cols < _VOCAB, lt, _NEG)
    logits_ref[v] = lt
    ltmax = jnp.max(lt, axis=1, keepdims=True)

    @pl.when(v == 0)
    def _():
        m_ref[...] = ltmax

    @pl.when(v > 0)
    def _():
        m_ref[...] = jnp.maximum(m_ref[...], ltmax)

    @pl.when(v == _NV - 1)
    def _finalize():
        neg_inf = jnp.float32(-jnp.inf)
        log2e = jnp.float32(1.4426950408889634)
        m = m_ref[...]

        def se(j, acc):
            ej = jnp.exp2(log2e * (logits_ref[j] - m))
            logits_ref[j] = ej
            for t in range(_TV // 128):
                acc = acc + ej[:, t * 128:(t + 1) * 128]
            return acc

        acc = lax.fori_loop(0, _NV, se, jnp.zeros((B, 128), jnp.float32))
        c = acc[:, 0:8]
        for t in range(1, 16):
            c = c + acc[:, 8 * t:8 * (t + 1)]
        u = c[:, 0:4] + c[:, 4:8]
        w = u[:, 0:2] + u[:, 2:4]
        s = w[:, 0:1] + w[:, 1:2]
        r = pl.reciprocal(s, approx=True)

        kiota = lax.broadcasted_iota(jnp.int32, (B, _K), 1)

        _DEP = 12
        lane32 = lax.broadcasted_iota(jnp.int32, (B, 32), 1)

        def build(j, carry):
            rs, cs = carry
            cj = lane + j * _TV
            cur = jnp.where(cj < _VOCAB, logits_ref[j] * r, -1.0)
            logits_ref[j] = cur
            rs2, cs2 = [], []
            for d in range(_DEP):
                mj = jnp.max(cur, axis=1, keepdims=True)
                aj = jnp.min(jnp.where(cur == mj, lane, _TV), axis=1,
                             keepdims=True)
                gcol = aj + j * _TV
                rs2.append(jnp.where(lane32 == j, mj, rs[d]))
                cs2.append(jnp.where(lane32 == j, gcol, cs[d]))
                if d + 1 < _DEP:
                    cur = jnp.where(lane == aj, neg_inf, cur)
            return tuple(rs2), tuple(cs2)

        rs0 = tuple(jnp.full((B, 32), neg_inf, jnp.float32)
                    for _ in range(_DEP))
        cs0 = tuple(jnp.zeros((B, 32), jnp.int32) for _ in range(_DEP))
        rs, cs = lax.fori_loop(0, _NV, build, (rs0, cs0))

        def pick_fast(k, carry):
            vals, colsb, cnt = carry
            cur = jnp.full((B, 32), neg_inf, jnp.float32)
            acol = jnp.zeros((B, 32), jnp.int32)
            for d in range(_DEP):
                hit = cnt == d
                cur = jnp.where(hit, rs[d], cur)
                acol = jnp.where(hit, cs[d], acol)
            cur = jnp.where(lane32 < _NV, cur, neg_inf)
            cm = jnp.max(cur, axis=1, keepdims=True)
            cidx = jnp.min(jnp.where(cur == cm, lane32, 32), axis=1,
                           keepdims=True)
            hitl = lane32 == cidx
            ca = jnp.sum(jnp.where(hitl, acol, 0), axis=1, keepdims=True)
            cnt = jnp.where(hitl, cnt + 1, cnt)
            sel = kiota == k
            vals = jnp.where(sel, cm, vals)
            colsb = jnp.where(sel, ca, colsb)
            return vals, colsb, cnt

        vals_f, colsb_f, cnt = lax.fori_loop(
            0, _K, pick_fast,
            (jnp.zeros((B, _K), jnp.float32),
             jnp.zeros((B, _K), jnp.int32),
             jnp.zeros((B, 32), jnp.int32)))
        overflow = jnp.any(cnt >= _DEP)

        def pick_slow(_):
            def pick(k, carry):
                vals, colsb, prev = carry

                def scan(j, c):
                    cm, ca = c
                    xj = logits_ref[j]
                    xj = jnp.where(lane == prev - j * _TV, neg_inf, xj)
                    logits_ref[j] = xj
                    mj = jnp.max(xj, axis=1, keepdims=True)
                    aj = jnp.min(jnp.where(xj == mj, lane, _TV), axis=1,
                                 keepdims=True) + j * _TV
                    better = mj > cm
                    return jnp.maximum(cm, mj), jnp.where(better, aj, ca)

                cm, ca = lax.fori_loop(
                    0, _NV, scan,
                    (jnp.full((B, 1), neg_inf, jnp.float32),
                     jnp.zeros((B, 1), jnp.int32)))
                sel = kiota == k
                vals = jnp.where(sel, cm, vals)
                colsb = jnp.where(sel, ca, colsb)
                return vals, colsb, ca

            vals, colsb, _ = lax.fori_loop(
                0, _K, pick,
                (jnp.zeros((B, _K), jnp.float32),
                 jnp.zeros((B, _K), jnp.int32),
                 jnp.full((B, 1), -1, jnp.int32)))
            return vals, colsb

        vals, colsb = lax.cond(
            overflow, pick_slow, lambda _: (vals_f, colsb_f), 0)

        probs = vals
        y = jnp.log(probs + 1e-20) + g_ref[...]
        ym = jnp.max(y, axis=1, keepdims=True)
        ix = jnp.min(jnp.where(y == ym, kiota, _K), axis=1, keepdims=True)
        tok = jnp.sum(jnp.where(kiota == ix, colsb, 0), axis=1, keepdims=True)

        tok_ref[...] = tok
        probs_ref[...] = probs
        idx_ref[...] = colsb


def _tc_call(h, w_unembed, g):
    B = h.shape[0]
    return pl.pallas_call(
        _tc_body,
        grid=(_NV,),
        in_specs=[
            pl.BlockSpec((B, _D), lambda v: (0, 0)),
            pl.BlockSpec((_D, _TV), lambda v: (0, v)),
            pl.BlockSpec((B, _K), lambda v: (0, 0)),
        ],
        out_specs=[
            pl.BlockSpec((B, 1), lambda v: (0, 0)),
            pl.BlockSpec((B, _K), lambda v: (0, 0)),
            pl.BlockSpec((B, _K), lambda v: (0, 0)),
        ],
        out_shape=[
            jax.ShapeDtypeStruct((B, 1), jnp.int32),
            jax.ShapeDtypeStruct((B, _K), jnp.float32),
            jax.ShapeDtypeStruct((B, _K), jnp.int32),
        ],
        scratch_shapes=[pltpu.VMEM((_NV, B, _TV), jnp.float32),
                        pltpu.VMEM((B, 1), jnp.float32)],
        compiler_params=pltpu.CompilerParams(
            dimension_semantics=("arbitrary",)),
    )(h, w_unembed, g)


def kernel(idx, wte, w_unembed):
    last_idx = idx[:, -1]
    h = _sc_gather(last_idx, wte)
    g = jax.random.gumbel(jax.random.key(42), (idx.shape[0], _K), jnp.float32)
    next_token, topk_probs, topk_indices = _tc_call(h, w_unembed, g)
    return next_token, topk_probs, topk_indices


# speedup vs baseline: 10.2105x; 1.0042x over previous
import functools

import jax
import jax.numpy as jnp
from jax import lax
from jax.experimental import pallas as pl
from jax.experimental.pallas import tpu as pltpu
from jax.experimental.pallas import tpu_sc as plsc

_VOCAB = 50257
_D = 768
_K = 50
_TV = 2048
_NV = (_VOCAB + _TV - 1) // _TV
_NEG = -1e30
_DEP = 12


def _sc_gather(last_idx, wte):
    B = last_idx.shape[0]
    D = wte.shape[1]
    nwork = 8
    bpw = B // nwork
    mesh = plsc.VectorSubcoreMesh(core_axis_name="c", subcore_axis_name="s")

    @functools.partial(
        pl.kernel,
        mesh=mesh,
        out_type=jax.ShapeDtypeStruct((B, D), jnp.float32),
        scratch_types=[
            pltpu.VMEM((bpw,), jnp.int32),
            pltpu.VMEM((bpw, D), jnp.float32),
            pltpu.SemaphoreType.DMA,
        ],
    )
    def gather_kernel(idx_hbm, table_hbm, out_hbm, idx_v, rows_v, sem):
        nc = 2
        wid = lax.axis_index("s") * nc + lax.axis_index("c")

        @pl.when(wid < nwork)
        def _():
            base = wid * bpw
            pltpu.sync_copy(idx_hbm.at[pl.ds(base, bpw)], idx_v)
            pltpu.async_copy(table_hbm.at[idx_v], rows_v, sem).wait()
            pltpu.sync_copy(rows_v, out_hbm.at[pl.ds(base, bpw)])

    return gather_kernel(last_idx, wte)


def _tc_body(h_ref, w_ref, g_ref, tok_ref, probs_ref, idx_ref, logits_ref,
             m_ref, rl_ref, cl_ref):
    v = pl.program_id(0)
    B = h_ref.shape[0]

    lane = lax.broadcasted_iota(jnp.int32, (B, _TV), 1)
    cols = lane + v * _TV
    lt = jnp.dot(h_ref[...], w_ref[...], preferred_element_type=jnp.float32)
    lt = jnp.where(cols < _VOCAB, lt, _NEG)
    logits_ref[v] = lt
    ltmax = jnp.max(lt, axis=1, keepdims=True)

    @pl.when(v == 0)
    def _():
        m_ref[...] = ltmax

    @pl.when(v > 0)
    def _():
        m_ref[...] = jnp.maximum(m_ref[...], ltmax)

    @pl.when(v == _NV - 1)
    def _finalize():
        neg_inf = jnp.float32(-jnp.inf)
        log2e = jnp.float32(1.4426950408889634)
        m = m_ref[...]

        def se(j, acc):
            ej = jnp.exp2(log2e * (logits_ref[j] - m))
            logits_ref[j] = ej
            for t in range(_TV // 128):
                acc = acc + ej[:, t * 128:(t + 1) * 128]
            return acc

        acc = lax.fori_loop(0, _NV, se, jnp.zeros((B, 128), jnp.float32))
        c = acc[:, 0:8]
        for t in range(1, 16):
            c = c + acc[:, 8 * t:8 * (t + 1)]
        u = c[:, 0:4] + c[:, 4:8]
        w = u[:, 0:2] + u[:, 2:4]
        s = w[:, 0:1] + w[:, 1:2]
        r = pl.reciprocal(s, approx=True)

        kiota = lax.broadcasted_iota(jnp.int32, (B, _K), 1)

        lane32 = lax.broadcasted_iota(jnp.int32, (B, 32), 1)

        def build(j, _):
            cj = lane + j * _TV
            cur = jnp.where(cj < _VOCAB, logits_ref[j] * r, -1.0)
            logits_ref[j] = cur
            ins = lane32 == j
            for d in range(_DEP):
                mj = jnp.max(cur, axis=1, keepdims=True)
                aj = jnp.min(jnp.where(cur == mj, lane, _TV), axis=1,
                             keepdims=True)
                gcol = aj + j * _TV
                rl_ref[d] = jnp.where(ins, mj, rl_ref[d])
                cl_ref[d] = jnp.where(ins, gcol, cl_ref[d])
                if d + 1 < _DEP:
                    cur = jnp.where(lane == aj, neg_inf, cur)
            return 0

        lax.fori_loop(0, _NV, build, 0)

        def pick_fast(k, carry):
            vals, colsb, cnt = carry
            cur = jnp.full((B, 32), neg_inf, jnp.float32)
            acol = jnp.zeros((B, 32), jnp.int32)
            for d in range(_DEP):
                hit = cnt == d
                cur = jnp.where(hit, rl_ref[d], cur)
                acol = jnp.where(hit, cl_ref[d], acol)
            cur = jnp.where(lane32 < _NV, cur, neg_inf)
            cm = jnp.max(cur, axis=1, keepdims=True)
            cidx = jnp.min(jnp.where(cur == cm, lane32, 32), axis=1,
                           keepdims=True)
            hitl = lane32 == cidx
            ca = jnp.sum(jnp.where(hitl, acol, 0), axis=1, keepdims=True)
            cnt = jnp.where(hitl, cnt + 1, cnt)
            sel = kiota == k
            vals = jnp.where(sel, cm, vals)
            colsb = jnp.where(sel, ca, colsb)
            return vals, colsb, cnt

        vals_f, colsb_f, cnt = lax.fori_loop(
            0, _K, pick_fast,
            (jnp.zeros((B, _K), jnp.float32),
             jnp.zeros((B, _K), jnp.int32),
             jnp.zeros((B, 32), jnp.int32)))
        overflow = jnp.any(cnt >= _DEP)

        def pick_slow(_):
            def pick(k, carry):
                vals, colsb, prev = carry

                def scan(j, c):
                    cm, ca = c
                    xj = logits_ref[j]
                    xj = jnp.where(lane == prev - j * _TV, neg_inf, xj)
                    logits_ref[j] = xj
                    mj = jnp.max(xj, axis=1, keepdims=True)
                    aj = jnp.min(jnp.where(xj == mj, lane, _TV), axis=1,
                                 keepdims=True) + j * _TV
                    better = mj > cm
                    return jnp.maximum(cm, mj), jnp.where(better, aj, ca)

                cm, ca = lax.fori_loop(
                    0, _NV, scan,
                    (jnp.full((B, 1), neg_inf, jnp.float32),
                     jnp.zeros((B, 1), jnp.int32)))
                sel = kiota == k
                vals = jnp.where(sel, cm, vals)
                colsb = jnp.where(sel, ca, colsb)
                return vals, colsb, ca

            vals, colsb, _ = lax.fori_loop(
                0, _K, pick,
                (jnp.zeros((B, _K), jnp.float32),
                 jnp.zeros((B, _K), jnp.int32),
                 jnp.full((B, 1), -1, jnp.int32)))
            return vals, colsb

        vals, colsb = lax.cond(
            overflow, pick_slow, lambda _: (vals_f, colsb_f), 0)

        probs = vals
        y = jnp.log(probs + 1e-20) + g_ref[...]
        ym = jnp.max(y, axis=1, keepdims=True)
        ix = jnp.min(jnp.where(y == ym, kiota, _K), axis=1, keepdims=True)
        tok = jnp.sum(jnp.where(kiota == ix, colsb, 0), axis=1, keepdims=True)

        tok_ref[...] = tok
        probs_ref[...] = probs
        idx_ref[...] = colsb


def _tc_call(h, w_unembed, g):
    B = h.shape[0]
    return pl.pallas_call(
        _tc_body,
        grid=(_NV,),
        in_specs=[
            pl.BlockSpec((B, _D), lambda v: (0, 0)),
            pl.BlockSpec((_D, _TV), lambda v: (0, v)),
            pl.BlockSpec((B, _K), lambda v: (0, 0)),
        ],
        out_specs=[
            pl.BlockSpec((B, 1), lambda v: (0, 0)),
            pl.BlockSpec((B, _K), lambda v: (0, 0)),
            pl.BlockSpec((B, _K), lambda v: (0, 0)),
        ],
        out_shape=[
            jax.ShapeDtypeStruct((B, 1), jnp.int32),
            jax.ShapeDtypeStruct((B, _K), jnp.float32),
            jax.ShapeDtypeStruct((B, _K), jnp.int32),
        ],
        scratch_shapes=[pltpu.VMEM((_NV, B, _TV), jnp.float32),
                        pltpu.VMEM((B, 1), jnp.float32),
                        pltpu.VMEM((_DEP, B, 32), jnp.float32),
                        pltpu.VMEM((_DEP, B, 32), jnp.int32)],
        compiler_params=pltpu.CompilerParams(
            dimension_semantics=("arbitrary",)),
    )(h, w_unembed, g)


def kernel(idx, wte, w_unembed):
    last_idx = idx[:, -1]
    h = _sc_gather(last_idx, wte)
    g = jax.random.gumbel(jax.random.key(42), (idx.shape[0], _K), jnp.float32)
    next_token, topk_probs, topk_indices = _tc_call(h, w_unembed, g)
    return next_token, topk_probs, topk_indices
